# baseline (speedup 1.0000x reference)
import functools

import jax
import jax.numpy as jnp
from jax import lax
from jax.experimental import pallas as pl
from jax.experimental.pallas import tpu as pltpu
from jax.experimental.pallas import tpu_sc as plsc

_DIM = 32
_NC = 2
_NS = 16
_NW = _NC * _NS
_CHUNK = 128


@functools.lru_cache
def _build(B):
    assert B % (_NW * _CHUNK) == 0
    b_per_w = B // _NW
    n_chunks = b_per_w // _CHUNK
    mesh = plsc.VectorSubcoreMesh(
        core_axis_name="c", subcore_axis_name="s",
        num_cores=_NC, num_subcores=_NS)

    @functools.partial(
        pl.kernel,
        mesh=mesh,
        out_type=jax.ShapeDtypeStruct((B, _DIM), jnp.float32),
        scratch_types=[
            pltpu.VMEM((n_chunks, _CHUNK), jnp.int32),
            pltpu.VMEM((_CHUNK, _DIM), jnp.float32),
            pltpu.SemaphoreType.DMA,
        ],
        compiler_params=pltpu.CompilerParams(use_tc_tiling_on_sc=False),
    )
    def emb(idx_hbm, table_hbm, out_hbm, idx_v, rows, sem):
        wid = lax.axis_index("s") * _NC + lax.axis_index("c")
        base = wid * b_per_w
        pltpu.sync_copy(idx_hbm.at[wid], idx_v)

        @pl.loop(0, n_chunks)
        def _(j):
            pltpu.async_copy(table_hbm.at[idx_v.at[j]], rows, sem).wait()
            pltpu.sync_copy(rows, out_hbm.at[pl.ds(base + j * _CHUNK, _CHUNK)])

    return emb


def kernel(x, weight):
    orig_shape = x.shape
    flat = x.reshape(-1).astype(jnp.int32)
    b = flat.shape[0]
    idx3d = flat.reshape(_NW, b // (_NW * _CHUNK), _CHUNK)
    out = _build(b)(idx3d, weight)
    return out.reshape(orig_shape + (weight.shape[-1],))


# speedup vs baseline: 1.0429x; 1.0429x over previous
import functools

import jax
import jax.numpy as jnp
from jax import lax
from jax.experimental import pallas as pl
from jax.experimental.pallas import tpu as pltpu
from jax.experimental.pallas import tpu_sc as plsc

_DIM = 32
_NC = 2
_NS = 16
_NW = _NC * _NS
_CHUNK = 128
_K = 5
_SUPER = _CHUNK * _K


@functools.lru_cache
def _build(B):
    assert B % (_NW * _SUPER) == 0
    b_per_w = B // _NW
    n_chunks = b_per_w // _CHUNK
    n_super = b_per_w // _SUPER
    assert n_super % 2 == 0 and n_super >= 4
    mesh = plsc.VectorSubcoreMesh(
        core_axis_name="c", subcore_axis_name="s",
        num_cores=_NC, num_subcores=_NS)

    @functools.partial(
        pl.kernel,
        mesh=mesh,
        out_type=jax.ShapeDtypeStruct((B, _DIM), jnp.float32),
        scratch_types=[
            pltpu.VMEM((n_chunks, _CHUNK), jnp.int32),
        ] + [pltpu.VMEM((_CHUNK, _DIM), jnp.float32) for _ in range(2 * _K)] + [
            pltpu.SemaphoreType.DMA,
            pltpu.SemaphoreType.DMA,
            pltpu.SemaphoreType.DMA,
        ],
        compiler_params=pltpu.CompilerParams(use_tc_tiling_on_sc=False),
    )
    def emb(idx_hbm, table_hbm, out_hbm, idx_v, *rest):
        rows = rest[:2 * _K]
        isem, gsem0, gsem1 = rest[2 * _K:]
        gsems = (gsem0, gsem1)
        wid = lax.axis_index("s") * _NC + lax.axis_index("c")
        base = wid * b_per_w
        pltpu.async_copy(idx_hbm.at[wid], idx_v, isem).wait()

        def cp(s, k, grp):
            return pltpu.make_async_copy(
                table_hbm.at[idx_v.at[s * _K + k]],
                rows[grp * _K + k],
                gsems[grp])

        def fire(s, grp):
            for k in range(_K):
                cp(s, k, grp).start()

        def drain_write(s, grp):
            for k in range(_K):
                cp(s, k, grp).wait()
            for k in range(_K):
                pltpu.sync_copy(
                    rows[grp * _K + k],
                    out_hbm.at[pl.ds(base + (s * _K + k) * _CHUNK, _CHUNK)])

        fire(0, 0)

        @pl.loop(0, n_super - 2, step=2)
        def _(s):
            fire(s + 1, 1)
            drain_write(s, 0)
            fire(s + 2, 0)
            drain_write(s + 1, 1)

        s_last = n_super - 2
        fire(s_last + 1, 1)
        drain_write(s_last, 0)
        drain_write(s_last + 1, 1)

    return emb


def kernel(x, weight):
    orig_shape = x.shape
    flat = x.reshape(-1).astype(jnp.int32)
    b = flat.shape[0]
    idx3d = flat.reshape(_NW, b // (_NW * _CHUNK), _CHUNK)
    out = _build(b)(idx3d, weight)
    return out.reshape(orig_shape + (weight.shape[-1],))


# speedup vs baseline: 1.1926x; 1.1436x over previous
import functools

import jax
import jax.numpy as jnp
from jax import lax
from jax.experimental import pallas as pl
from jax.experimental.pallas import tpu as pltpu
from jax.experimental.pallas import tpu_sc as plsc

_DIM = 32
_NC = 2
_NS = 16
_NW = _NC * _NS
_CHUNK = 128
_L = 16


@functools.lru_cache
def _build(B, V):
    assert B % (_NW * _CHUNK) == 0 and V % 4 == 0
    b_per_w = B // _NW
    n_chunks = b_per_w // _CHUNK
    assert n_chunks % 2 == 0 and n_chunks >= 4
    mesh = plsc.VectorSubcoreMesh(
        core_axis_name="c", subcore_axis_name="s",
        num_cores=_NC, num_subcores=_NS)

    @functools.partial(
        pl.kernel,
        mesh=mesh,
        out_type=jax.ShapeDtypeStruct((B * _DIM,), jnp.float32),
        scratch_types=[
            pltpu.VMEM((b_per_w,), jnp.int32),
            pltpu.VMEM((b_per_w,), jnp.int32),
            pltpu.VMEM((_CHUNK, 128), jnp.float32),
            pltpu.VMEM((_CHUNK, 128), jnp.float32),
            pltpu.VMEM((_CHUNK * _DIM,), jnp.float32),
            pltpu.VMEM((_CHUNK * _DIM,), jnp.float32),
            pltpu.SemaphoreType.DMA,
            pltpu.SemaphoreType.DMA,
            pltpu.SemaphoreType.DMA,
        ],
    )
    def emb(idx_hbm, table_hbm, out_hbm, idx_v, q_v, g0, g1, o0, o1,
            isem, gsem0, gsem1):
        gbuf = (g0, g1)
        obuf = (o0, o1)
        gsems = (gsem0, gsem1)
        wid = lax.axis_index("s") * _NC + lax.axis_index("c")
        base = wid * b_per_w
        pltpu.async_copy(idx_hbm.at[pl.ds(base, b_per_w)], idx_v, isem).wait()

        @pl.loop(0, b_per_w // _L, unroll=8)
        def _(k):
            v = idx_v[pl.ds(k * _L, _L)]
            q_v[pl.ds(k * _L, _L)] = jax.lax.shift_right_logical(v, 2)

        def cp(c, b):
            return pltpu.make_async_copy(
                table_hbm.at[q_v.at[pl.ds(c * _CHUNK, _CHUNK)]],
                gbuf[b], gsems[b])

        def extract(c, b):
            g, o = gbuf[b], obuf[b]

            @pl.loop(0, _CHUNK // _L)
            def _(gi):
                vidx = idx_v[pl.ds(c * _CHUNK + gi * _L, _L)]
                voff = jax.lax.shift_left(vidx & 3, 5)
                for l in range(_L):
                    i = gi * _L + l
                    off = voff[l]
                    o[pl.ds(i * _DIM, _L)] = g[i, pl.ds(off, _L)]
                    o[pl.ds(i * _DIM + _L, _L)] = g[i, pl.ds(off + _L, _L)]

        def write(c, b):
            pltpu.sync_copy(
                obuf[b],
                out_hbm.at[pl.ds((base + c * _CHUNK) * _DIM, _CHUNK * _DIM)])

        cp(0, 0).start()
        cp(1, 1).start()

        @pl.loop(0, n_chunks - 2, step=2)
        def _(c):
            for b in range(2):
                cp(c + b, b).wait()
                extract(c + b, b)
                cp(c + b + 2, b).start()
                write(c + b, b)

        for b in range(2):
            c = n_chunks - 2 + b
            cp(c, b).wait()
            extract(c, b)
            write(c, b)

    return emb


def kernel(x, weight):
    orig_shape = x.shape
    flat = x.reshape(-1).astype(jnp.int32)
    b = flat.shape[0]
    v = weight.shape[0]
    table = weight.reshape(v // 4, 128)
    out = _build(b, v)(flat, table)
    return out.reshape(orig_shape + (weight.shape[-1],))


# speedup vs baseline: 1.1933x; 1.0006x over previous
import functools

import jax
import jax.numpy as jnp
from jax import lax
from jax.experimental import pallas as pl
from jax.experimental.pallas import tpu as pltpu
from jax.experimental.pallas import tpu_sc as plsc

_DIM = 32
_NC = 2
_NS = 16
_NW = _NC * _NS
_CHUNK = 128
_L = 16


@functools.lru_cache
def _build(B, V):
    assert B % (_NW * _CHUNK) == 0 and V % 4 == 0
    b_per_w = B // _NW
    n_chunks = b_per_w // _CHUNK
    assert n_chunks % 2 == 0 and n_chunks >= 4
    mesh = plsc.VectorSubcoreMesh(
        core_axis_name="c", subcore_axis_name="s",
        num_cores=_NC, num_subcores=_NS)

    @functools.partial(
        pl.kernel,
        mesh=mesh,
        out_type=jax.ShapeDtypeStruct((B * _DIM,), jnp.float32),
        scratch_types=[
            pltpu.VMEM((b_per_w,), jnp.int32),
            pltpu.VMEM((b_per_w,), jnp.int32),
            pltpu.VMEM((_CHUNK, 128), jnp.float32),
            pltpu.VMEM((_CHUNK, 128), jnp.float32),
            pltpu.VMEM((_CHUNK * _DIM,), jnp.float32),
            pltpu.VMEM((_CHUNK * _DIM,), jnp.float32),
            pltpu.SemaphoreType.DMA,
            pltpu.SemaphoreType.DMA,
            pltpu.SemaphoreType.DMA,
        ],
        compiler_params=pltpu.CompilerParams(use_tc_tiling_on_sc=True),
    )
    def emb(idx_hbm, table_hbm, out_hbm, idx_v, q_v, g0, g1, o0, o1,
            isem, gsem0, gsem1):
        gbuf = (g0, g1)
        obuf = (o0, o1)
        gsems = (gsem0, gsem1)
        wid = lax.axis_index("s") * _NC + lax.axis_index("c")
        base = wid * b_per_w
        pltpu.async_copy(idx_hbm.at[pl.ds(base, b_per_w)], idx_v, isem).wait()

        @pl.loop(0, b_per_w // _L, unroll=8)
        def _(k):
            v = idx_v[pl.ds(k * _L, _L)]
            q_v[pl.ds(k * _L, _L)] = jax.lax.shift_right_logical(v, 2)

        def cp(c, b):
            return pltpu.make_async_copy(
                table_hbm.at[q_v.at[pl.ds(c * _CHUNK, _CHUNK)]],
                gbuf[b], gsems[b])

        def extract(c, b):
            g, o = gbuf[b], obuf[b]

            @pl.loop(0, _CHUNK // _L)
            def _(gi):
                vidx = idx_v[pl.ds(c * _CHUNK + gi * _L, _L)]
                voff = jax.lax.shift_left(vidx & 3, 5)
                for l in range(_L):
                    i = gi * _L + l
                    off = voff[l]
                    o[pl.ds(i * _DIM, _L)] = g[i, pl.ds(off, _L)]
                    o[pl.ds(i * _DIM + _L, _L)] = g[i, pl.ds(off + _L, _L)]

        def write(c, b):
            pltpu.sync_copy(
                obuf[b],
                out_hbm.at[pl.ds((base + c * _CHUNK) * _DIM, _CHUNK * _DIM)])

        cp(0, 0).start()
        cp(1, 1).start()

        @pl.loop(0, n_chunks - 2, step=2)
        def _(c):
            for b in range(2):
                cp(c + b, b).wait()
                extract(c + b, b)
                cp(c + b + 2, b).start()
                write(c + b, b)

        for b in range(2):
            c = n_chunks - 2 + b
            cp(c, b).wait()
            extract(c, b)
            write(c, b)

    return emb


def kernel(x, weight):
    orig_shape = x.shape
    flat = x.reshape(-1).astype(jnp.int32)
    b = flat.shape[0]
    v = weight.shape[0]
    table = weight.reshape(v // 4, 128)
    out = _build(b, v)(flat, table)
    return out.reshape(orig_shape + (weight.shape[-1],))


# speedup vs baseline: 1.1968x; 1.0029x over previous
import functools

import jax
import jax.numpy as jnp
from jax import lax
from jax.experimental import pallas as pl
from jax.experimental.pallas import tpu as pltpu
from jax.experimental.pallas import tpu_sc as plsc

_DIM = 32
_NC = 2
_NS = 16
_NW = _NC * _NS
_CHUNK = 128
_L = 16


@functools.lru_cache
def _build(S, B):
    assert B == _NW * _CHUNK and S % 2 == 0
    mesh = plsc.VectorSubcoreMesh(
        core_axis_name="c", subcore_axis_name="s",
        num_cores=_NC, num_subcores=_NS)

    @functools.partial(
        pl.kernel,
        mesh=mesh,
        out_type=jax.ShapeDtypeStruct((S, _DIM, B), jnp.float32),
        scratch_types=[
            pltpu.VMEM((8, _CHUNK), jnp.int32),
            pltpu.VMEM((8, _CHUNK), jnp.int32),
            pltpu.VMEM((_CHUNK,), jnp.int32),
            pltpu.VMEM((_CHUNK,), jnp.int32),
            pltpu.VMEM((_CHUNK, 128), jnp.float32),
            pltpu.VMEM((_CHUNK, 128), jnp.float32),
            pltpu.VMEM((_DIM, _CHUNK), jnp.float32),
            pltpu.VMEM((_DIM, _CHUNK), jnp.float32),
            pltpu.SemaphoreType.DMA,
            pltpu.SemaphoreType.DMA,
        ],
        compiler_params=pltpu.CompilerParams(
            use_tc_tiling_on_sc=True, needs_layout_passes=False),
    )
    def emb(idx_hbm, table_hbm, out_hbm, i0, i1, q0, q1, g0, g1, o0, o1,
            gsem0, gsem1):
        ibuf = (i0, i1)
        qbuf = (q0, q1)
        gbuf = (g0, g1)
        obuf = (o0, o1)
        gsems = (gsem0, gsem1)
        wid = lax.axis_index("s") * _NC + lax.axis_index("c")
        blk = lax.div(wid, 8)
        sub = lax.rem(wid, 8)
        clo = lax.iota(jnp.int32, _L)
        chi = clo + _L

        def load_idx(s, b):
            pltpu.sync_copy(idx_hbm.at[s, blk], ibuf[b])

        def comp_q(b):
            for gi in range(_CHUNK // _L):
                v = ibuf[b][sub, pl.ds(gi * _L, _L)]
                qbuf[b][pl.ds(gi * _L, _L)] = lax.shift_right_logical(v, 2)

        def cp(b):
            return pltpu.make_async_copy(
                table_hbm.at[qbuf[b]], gbuf[b], gsems[b])

        def extract(b):
            g, o = gbuf[b], obuf[b]

            @pl.loop(0, _CHUNK // _L)
            def _(gi):
                vidx = ibuf[b][sub, pl.ds(gi * _L, _L)]
                voff = lax.shift_left(vidx & 3, 5)
                for l in range(_L):
                    i = gi * _L + l
                    iv = jnp.full((_L,), i, jnp.int32)
                    off = voff[l]
                    plsc.store_scatter(o, [clo, iv], g[i, pl.ds(off, _L)])
                    plsc.store_scatter(o, [chi, iv], g[i, pl.ds(off + _L, _L)])

        def write(s, b):
            pltpu.sync_copy(
                obuf[b], out_hbm.at[s, :, pl.ds(wid * _CHUNK, _CHUNK)])

        for b in range(2):
            load_idx(b, b)
            comp_q(b)
            cp(b).start()

        @pl.loop(0, S - 2, step=2)
        def _(s):
            for b in range(2):
                cp(b).wait()
                extract(b)
                write(s + b, b)
                load_idx(s + b + 2, b)
                comp_q(b)
                cp(b).start()

        for b in range(2):
            cp(b).wait()
            extract(b)
            write(S - 2 + b, b)

    return emb


def kernel(x, weight):
    orig_shape = x.shape
    v, dim = weight.shape
    s = x.shape[-1]
    b = x.size // s
    xt = x.T.astype(jnp.int32).reshape(s, b // _CHUNK // 8, 8, _CHUNK)
    table = weight.reshape(v // 4, dim * 4)
    out_phys = _build(s, b)(xt, table)
    return jnp.transpose(out_phys, (2, 0, 1)).reshape(orig_shape + (dim,))
